# Initial kernel scaffold; baseline (speedup 1.0000x reference)
import jax
import jax.numpy as jnp
from jax.experimental import pallas as pl


def kernel(x, y, Wq, bq, Wk, bk, Wv, bv):
    raise NotImplementedError("write your pallas kernel here")


# speedup vs baseline: 1.5536x; 1.5536x over previous
import jax
import jax.numpy as jnp
from jax.experimental import pallas as pl


def _kv_proj_kernel(y_ref, wkt_ref, bk_ref, wvt_ref, bv_ref, k_ref, v_ref):
    yb = y_ref[0]
    k_ref[0] = jnp.dot(yb, wkt_ref[...],
                       preferred_element_type=jnp.float32) + bk_ref[...]
    v_ref[0] = jnp.dot(yb, wvt_ref[...],
                       preferred_element_type=jnp.float32) + bv_ref[...]


def _attn_kernel(x_ref, wqt_ref, bq_ref, k_ref, v_ref, o_ref):
    xb = x_ref[0]
    q = jnp.dot(xb, wqt_ref[...],
                preferred_element_type=jnp.float32) + bq_ref[...]
    k = k_ref[0]
    s = jax.lax.dot_general(q, k, (((1,), (1,)), ((), ())),
                            preferred_element_type=jnp.float32)
    m = jnp.max(s, axis=-1, keepdims=True)
    p = jnp.exp(s - m)
    l = jnp.sum(p, axis=-1, keepdims=True)
    o = jnp.dot(p, v_ref[0], preferred_element_type=jnp.float32)
    o_ref[0] = o / l + xb


def kernel(x, y, Wq, bq, Wk, bk, Wv, bv):
    B, SX, D = x.shape
    SY = y.shape[1]
    bs = min(512, SY)
    bq_rows = min(512, SX)

    wqt = Wq.T
    wkt = Wk.T
    wvt = Wv.T
    bq2 = bq.reshape(1, D)
    bk2 = bk.reshape(1, D)
    bv2 = bv.reshape(1, D)

    k, v = pl.pallas_call(
        _kv_proj_kernel,
        grid=(B, SY // bs),
        in_specs=[
            pl.BlockSpec((1, bs, D), lambda b, i: (b, i, 0)),
            pl.BlockSpec((D, D), lambda b, i: (0, 0)),
            pl.BlockSpec((1, D), lambda b, i: (0, 0)),
            pl.BlockSpec((D, D), lambda b, i: (0, 0)),
            pl.BlockSpec((1, D), lambda b, i: (0, 0)),
        ],
        out_specs=[
            pl.BlockSpec((1, bs, D), lambda b, i: (b, i, 0)),
            pl.BlockSpec((1, bs, D), lambda b, i: (b, i, 0)),
        ],
        out_shape=[
            jax.ShapeDtypeStruct((B, SY, D), jnp.float32),
            jax.ShapeDtypeStruct((B, SY, D), jnp.float32),
        ],
    )(y, wkt, bk2, wvt, bv2)

    out = pl.pallas_call(
        _attn_kernel,
        grid=(B, SX // bq_rows),
        in_specs=[
            pl.BlockSpec((1, bq_rows, D), lambda b, i: (b, i, 0)),
            pl.BlockSpec((D, D), lambda b, i: (0, 0)),
            pl.BlockSpec((1, D), lambda b, i: (0, 0)),
            pl.BlockSpec((1, SY, D), lambda b, i: (b, 0, 0)),
            pl.BlockSpec((1, SY, D), lambda b, i: (b, 0, 0)),
        ],
        out_specs=pl.BlockSpec((1, bq_rows, D), lambda b, i: (b, i, 0)),
        out_shape=jax.ShapeDtypeStruct((B, SX, D), jnp.float32),
    )(x, wqt, bq2, k, v)
    return out


# speedup vs baseline: 1.7184x; 1.1060x over previous
import jax
import jax.numpy as jnp
from jax.experimental import pallas as pl
from jax.experimental.pallas import tpu as pltpu


def _fused_kernel(x_ref, y_ref, wqt_ref, bq_ref, wkt_ref, bk_ref,
                  wvt_ref, bv_ref, o_ref, k_scr, v_scr):
    i = pl.program_id(1)

    @pl.when(i == 0)
    def _project_kv():
        yb = y_ref[0]
        k_scr[...] = jnp.dot(yb, wkt_ref[...],
                             preferred_element_type=jnp.float32) + bk_ref[...]
        v_scr[...] = jnp.dot(yb, wvt_ref[...],
                             preferred_element_type=jnp.float32) + bv_ref[...]

    @pl.when(i > 0)
    def _attend():
        xb = x_ref[0]
        q = jnp.dot(xb, wqt_ref[...],
                    preferred_element_type=jnp.float32) + bq_ref[...]
        s = jax.lax.dot_general(q, k_scr[...], (((1,), (1,)), ((), ())),
                                preferred_element_type=jnp.float32)
        m = jnp.max(s, axis=-1, keepdims=True)
        p = jnp.exp(s - m)
        l = jnp.sum(p, axis=-1, keepdims=True)
        o = jnp.dot(p, v_scr[...], preferred_element_type=jnp.float32)
        o_ref[0] = o / l + xb


def kernel(x, y, Wq, bq, Wk, bk, Wv, bv):
    B, SX, D = x.shape
    SY = y.shape[1]
    bq_rows = min(512, SX)
    nq = SX // bq_rows

    wqt = Wq.T
    wkt = Wk.T
    wvt = Wv.T
    bq2 = bq.reshape(1, D)
    bk2 = bk.reshape(1, D)
    bv2 = bv.reshape(1, D)

    def qi(b, i):
        return (b, jnp.maximum(i - 1, 0), 0)

    out = pl.pallas_call(
        _fused_kernel,
        grid=(B, nq + 1),
        in_specs=[
            pl.BlockSpec((1, bq_rows, D), qi),
            pl.BlockSpec((1, SY, D), lambda b, i: (b, 0, 0)),
            pl.BlockSpec((D, D), lambda b, i: (0, 0)),
            pl.BlockSpec((1, D), lambda b, i: (0, 0)),
            pl.BlockSpec((D, D), lambda b, i: (0, 0)),
            pl.BlockSpec((1, D), lambda b, i: (0, 0)),
            pl.BlockSpec((D, D), lambda b, i: (0, 0)),
            pl.BlockSpec((1, D), lambda b, i: (0, 0)),
        ],
        out_specs=pl.BlockSpec((1, bq_rows, D), qi),
        out_shape=jax.ShapeDtypeStruct((B, SX, D), jnp.float32),
        scratch_shapes=[
            pltpu.VMEM((SY, D), jnp.float32),
            pltpu.VMEM((SY, D), jnp.float32),
        ],
    )(x, y, wqt, bq2, wkt, bk2, wvt, bv2)
    return out
